# baseline (speedup 1.0000x reference)
import functools

import jax
import jax.numpy as jnp
from jax import lax
from jax.experimental import pallas as pl
from jax.experimental.pallas import tpu as pltpu
from jax.experimental.pallas import tpu_sc as plsc

_NC = 2
_NS = 16
_NW = _NC * _NS
_BCHUNK = 4
_NBUF = 4
_AHEAD = _NBUF - 1


@functools.lru_cache(maxsize=None)
def _make_gather(batch: int, hist: int, dim: int):
    assert batch % (_NW * _BCHUNK) == 0
    bpw = batch // _NW
    cpw = bpw // _BCHUNK
    npw = bpw * hist
    chunk_idx = _BCHUNK * hist
    assert cpw > _NBUF

    mesh = plsc.VectorSubcoreMesh(core_axis_name="c", subcore_axis_name="s")

    @functools.partial(
        pl.kernel,
        mesh=mesh,
        out_type=jax.ShapeDtypeStruct((batch, hist, dim), jnp.float32),
        scratch_types=[
            pltpu.VMEM((npw,), jnp.int32),
            pltpu.VMEM((_NBUF, chunk_idx, dim), jnp.float32),
            pltpu.SemaphoreType.DMA,
            pltpu.SemaphoreType.DMA,
        ],
        compiler_params=pltpu.CompilerParams(use_tc_tiling_on_sc=False),
    )
    def gather(weight_hbm, idx_hbm, out_hbm, idx_v, rows_v, gsem, wsem):
        c = lax.axis_index("c")
        s = lax.axis_index("s")
        wid = s * _NC + c
        row_base = wid * bpw
        pltpu.sync_copy(idx_hbm.at[pl.ds(wid * npw, npw)], idx_v)

        def fire_gather(j, b):
            pltpu.async_copy(
                weight_hbm.at[idx_v.at[pl.ds(j * chunk_idx, chunk_idx)]],
                rows_v.at[b],
                gsem,
            )

        def fire_writes(j, b):
            for k in range(_BCHUNK):
                pltpu.async_copy(
                    rows_v.at[b, pl.ds(k * hist, hist)],
                    out_hbm.at[row_base + j * _BCHUNK + k],
                    wsem,
                )

        def wait_writes(j, b):
            for k in range(_BCHUNK):
                pltpu.make_async_copy(
                    rows_v.at[b, pl.ds(k * hist, hist)],
                    out_hbm.at[row_base + j * _BCHUNK + k],
                    wsem,
                ).wait()

        for b in range(_AHEAD):
            fire_gather(b, b)

        def body(j, carry):
            b = lax.rem(j, _NBUF)
            jf = j + _AHEAD

            @pl.when(jf < cpw)
            def _():
                @pl.when(j >= 1)
                def _():
                    wait_writes(j - 1, lax.rem(j - 1, _NBUF))

                fire_gather(jf, lax.rem(jf, _NBUF))

            pltpu.make_async_copy(
                weight_hbm.at[idx_v.at[pl.ds(j * chunk_idx, chunk_idx)]],
                rows_v.at[b],
                gsem,
            ).wait()
            fire_writes(j, b)
            return carry

        lax.fori_loop(0, cpw, body, 0)

        for i in range(_NBUF):
            j = cpw - _NBUF + i
            wait_writes(j, j % _NBUF)

    return gather


def kernel(input_, weight):
    batch, hist = input_.shape
    dim = weight.shape[1]
    idx = input_.reshape(batch * hist).astype(jnp.int32)
    return _make_gather(batch, hist, dim)(weight, idx)
